# baseline (speedup 1.0000x reference)
import functools

import jax
import jax.numpy as jnp
from jax import lax
from jax.experimental import pallas as pl
from jax.experimental.pallas import tpu as pltpu
from jax.experimental.pallas import tpu_sc as plsc

B = 4096
T = 50
NSF = 26
NDF = 5
V = 100000
D = 16
NRS = 4
NRD = 8
CS = NSF * D
COUT = CS + NRS + NDF * D + NRD

NW = 32
BPW = B // NW
SCB = 32
SROWS = SCB * CS
NSCH = BPW // SCB

DCR = 256
DROWS = DCR * NDF
KSPLIT = 10


def _sc_gather(fsc_t, fdc_seg, ws_cols, wd_flat, t0, nt, with_static):
    mesh = plsc.VectorSubcoreMesh(core_axis_name="c", subcore_axis_name="s")
    rpw = nt * B // NW
    nch = rpw // DCR

    out_type = [jax.ShapeDtypeStruct((nt * B * NDF, D), jnp.float32)]
    scratch = [
        pltpu.VMEM((NDF * DCR,), jnp.int32),
        pltpu.VMEM((DROWS // 128, 128), jnp.int32),
        pltpu.VMEM((DROWS, D), jnp.float32),
        pltpu.SemaphoreType.DMA,
    ]
    if with_static:
        out_type = [jax.ShapeDtypeStruct((B * CS,), jnp.float32)] + out_type
        scratch = [
            pltpu.VMEM((NSF * BPW,), jnp.int32),
            pltpu.VMEM((SROWS // 128, 128), jnp.int32),
            pltpu.VMEM((SROWS,), jnp.float32),
        ] + scratch

    @functools.partial(
        pl.kernel,
        mesh=mesh,
        compiler_params=pltpu.CompilerParams(
            use_tc_tiling_on_sc=False, needs_layout_passes=False),
        out_type=tuple(out_type),
        scratch_types=scratch,
    )
    def k(*refs):
        if with_static:
            (fsc_h, fdc_h, wsc_h, wd_h, outs_h, outd_h,
             sbuf, sidx, sdst, dbuf, didx, ddst, sem) = refs
        else:
            fdc_h, wd_h, outd_h, dbuf, didx, ddst, sem = refs
        w = lax.axis_index("s") * 2 + lax.axis_index("c")
        iota = lax.iota(jnp.int32, 16)

        if with_static:
            b0 = w * BPW
            hs = [
                pltpu.async_copy(fsc_h.at[pl.ds(i * B + b0, BPW)],
                                 sbuf.at[pl.ds(i * BPW, BPW)], sem)
                for i in range(NSF)
            ]
            for h in hs:
                h.wait()

            def s_chunk(ci, carry):
                def comp(q, c2):
                    p = q * 16 + iota
                    col = p % CS
                    bl = ci * SCB + p // CS
                    raw = plsc.load_gather(sbuf, [(col // D) * BPW + bl])
                    sidx[q // 8, pl.ds((q % 8) * 16, 16)] = col * V + raw
                    return c2
                lax.fori_loop(0, SROWS // 16, comp, 0)

                def s_gat(j, c2):
                    pltpu.async_copy(wsc_h.at[sidx.at[j]],
                                     sdst.at[pl.ds(j * 128, 128)], sem)
                    return c2
                lax.fori_loop(0, SROWS // 128, s_gat, 0)
                pltpu.make_async_copy(wsc_h.at[pl.ds(0, SROWS)], sdst,
                                      sem).wait()
                pltpu.async_copy(
                    sdst, outs_h.at[pl.ds((b0 + ci * SCB) * CS, SROWS)],
                    sem).wait()
                return carry
            lax.fori_loop(0, NSCH, s_chunk, 0)

        g0 = t0 * B + w * rpw
        l0 = w * rpw

        def d_chunk(ci, carry):
            off = ci * DCR
            hseg = [
                pltpu.async_copy(fdc_h.at[f, pl.ds(g0 + off, DCR)],
                                 dbuf.at[pl.ds(f * DCR, DCR)], sem)
                for f in range(NDF)
            ]
            for h in hseg:
                h.wait()

            def comp(q, c2):
                p = q * 16 + iota
                rr = p // NDF
                f = p % NDF
                raw = plsc.load_gather(dbuf, [f * DCR + rr])
                didx[q // 8, pl.ds((q % 8) * 16, 16)] = f * V + raw
                return c2
            lax.fori_loop(0, DROWS // 16, comp, 0)

            def d_gat(j, c2):
                pltpu.async_copy(wd_h.at[didx.at[j]],
                                 ddst.at[pl.ds(j * 128, 128)], sem)
                return c2
            lax.fori_loop(0, DROWS // 128, d_gat, 0)
            pltpu.make_async_copy(wd_h.at[pl.ds(0, DROWS)], ddst,
                                  sem).wait()
            pltpu.async_copy(
                ddst, outd_h.at[pl.ds((l0 + off) * NDF, DROWS)],
                sem).wait()
            return carry
        lax.fori_loop(0, nch, d_chunk, 0)

    if with_static:
        return k(fsc_t, fdc_seg, ws_cols, wd_flat)
    return k(fdc_seg, wd_flat)


def _tc_assemble(stat_emb, stat_real, dyn_emb, dyn_real, nt, t_off,
                 prev=None):
    def body(*refs):
        se_ref, sr_ref, de_ref, dr_ref, o_ref = refs[-5:]
        stat = jnp.concatenate([se_ref[...], sr_ref[...]], axis=-1)
        o_ref[...] = jnp.concatenate(
            [stat[None], de_ref[...], dr_ref[...]], axis=-1)

    in_specs = [
        pl.BlockSpec((B, CS), lambda i: (0, 0)),
        pl.BlockSpec((B, NRS), lambda i: (0, 0)),
        pl.BlockSpec((1, B, NDF * D), lambda i: (i, 0, 0)),
        pl.BlockSpec((1, B, NRD), lambda i: (i + t_off, 0, 0)),
    ]
    args = [stat_emb, stat_real, dyn_emb, dyn_real]
    kwargs = {}
    if prev is not None:
        in_specs = [pl.BlockSpec(memory_space=pl.ANY)] + in_specs
        args = [prev] + args
        kwargs["input_output_aliases"] = {0: 0}

    return pl.pallas_call(
        body,
        compiler_params=pltpu.CompilerParams(
            vmem_limit_bytes=100 * 1024 * 1024),
        grid=(nt,),
        in_specs=in_specs,
        out_specs=pl.BlockSpec((1, B, COUT), lambda i: (i + t_off, 0, 0)),
        out_shape=jax.ShapeDtypeStruct((T, B, COUT), jnp.float32),
        **kwargs,
    )(*args)


def kernel(feat_static_cat, feat_static_real, feat_dynamic_cat,
           feat_dynamic_real, W_static, W_dynamic):
    ws_cols = jnp.transpose(W_static, (0, 2, 1)).reshape(NSF * D * V)
    wd_flat = W_dynamic.reshape(NDF * V, D)
    fsc_t = jnp.transpose(feat_static_cat.astype(jnp.int32),
                          (1, 0)).reshape(NSF * B)
    fdc_seg = jnp.transpose(feat_dynamic_cat.astype(jnp.int32),
                            (2, 1, 0)).reshape(NDF, T * B)
    out_stat, dyn_a = _sc_gather(fsc_t, fdc_seg, ws_cols, wd_flat,
                                 0, KSPLIT, True)
    (dyn_b,) = _sc_gather(None, fdc_seg, None, wd_flat,
                          KSPLIT, T - KSPLIT, False)
    fdr_t = jnp.transpose(feat_dynamic_real, (1, 0, 2))
    se = out_stat.reshape(B, CS)
    out1 = _tc_assemble(se, feat_static_real,
                        dyn_a.reshape(KSPLIT, B, NDF * D), fdr_t,
                        KSPLIT, 0)
    out2 = _tc_assemble(se, feat_static_real,
                        dyn_b.reshape(T - KSPLIT, B, NDF * D), fdr_t,
                        T - KSPLIT, KSPLIT, prev=out1)
    return jnp.transpose(out2, (1, 0, 2))


# speedup vs baseline: 1.0457x; 1.0457x over previous
import functools

import jax
import jax.numpy as jnp
from jax import lax
from jax.experimental import pallas as pl
from jax.experimental.pallas import tpu as pltpu
from jax.experimental.pallas import tpu_sc as plsc

B = 4096
T = 50
NSF = 26
NDF = 5
V = 100000
D = 16
NRS = 4
NRD = 8
CS = NSF * D
COUT = CS + NRS + NDF * D + NRD

NW = 32
BPW = B // NW
SCB = 32
SROWS = SCB * CS
NSCH = BPW // SCB

RPW = (T * B) // NW
DCR = 400
DROWS = DCR * NDF
DPAD = 2048
NDCH = RPW // DCR


def _sc_gather(fsc_t, fdc_seg, ws_cols, wd_flat):
    mesh = plsc.VectorSubcoreMesh(core_axis_name="c", subcore_axis_name="s")

    @functools.partial(
        pl.kernel,
        mesh=mesh,
        compiler_params=pltpu.CompilerParams(
            use_tc_tiling_on_sc=False, needs_layout_passes=False),
        out_type=(
            jax.ShapeDtypeStruct((B * CS,), jnp.float32),
            jax.ShapeDtypeStruct((T * B * NDF, D), jnp.float32),
        ),
        scratch_types=[
            pltpu.VMEM((NSF * BPW,), jnp.int32),
            pltpu.VMEM((SROWS // 128, 128), jnp.int32),
            pltpu.VMEM((SROWS,), jnp.float32),
            pltpu.VMEM((NDF * DCR + 48,), jnp.int32),
            pltpu.VMEM((DPAD // 128, 128), jnp.int32),
            pltpu.VMEM((DPAD, D), jnp.float32),
            pltpu.SemaphoreType.DMA,
        ],
    )
    def k(fsc_h, fdc_h, wsc_h, wd_h, outs_h, outd_h,
          sbuf, sidx, sdst, dbuf, didx, ddst, sem):
        w = lax.axis_index("s") * 2 + lax.axis_index("c")
        iota = lax.iota(jnp.int32, 16)
        b0 = w * BPW
        r0 = w * RPW

        hs = [
            pltpu.async_copy(fsc_h.at[pl.ds(i * B + b0, BPW)],
                             sbuf.at[pl.ds(i * BPW, BPW)], sem)
            for i in range(NSF)
        ]
        for h in hs:
            h.wait()

        def s_chunk(ci, carry):
            def comp(q, c2):
                p = q * 16 + iota
                col = p % CS
                bl = ci * SCB + p // CS
                raw = plsc.load_gather(sbuf, [(col // D) * BPW + bl])
                sidx[q // 8, pl.ds((q % 8) * 16, 16)] = col * V + raw
                return c2
            lax.fori_loop(0, SROWS // 16, comp, 0)

            def s_gat(j, c2):
                pltpu.async_copy(wsc_h.at[sidx.at[j]],
                                 sdst.at[pl.ds(j * 128, 128)], sem)
                return c2
            lax.fori_loop(0, SROWS // 128, s_gat, 0)
            pltpu.make_async_copy(wsc_h.at[pl.ds(0, SROWS)], sdst,
                                  sem).wait()
            pltpu.async_copy(
                sdst, outs_h.at[pl.ds((b0 + ci * SCB) * CS, SROWS)],
                sem).wait()
            return carry
        lax.fori_loop(0, NSCH, s_chunk, 0)

        def d_chunk(ci, carry):
            off = ci * DCR
            hseg = [
                pltpu.async_copy(fdc_h.at[f, pl.ds(r0 + off, DCR)],
                                 dbuf.at[pl.ds(f * DCR, DCR)], sem)
                for f in range(NDF)
            ]
            for h in hseg:
                h.wait()

            def comp(q, c2):
                p = q * 16 + iota
                rr = jnp.minimum(p // NDF, DCR - 1)
                f = p % NDF
                raw = plsc.load_gather(dbuf, [f * DCR + rr])
                didx[q // 8, pl.ds((q % 8) * 16, 16)] = f * V + raw
                return c2
            lax.fori_loop(0, DPAD // 16, comp, 0)

            def d_gat(j, c2):
                pltpu.async_copy(wd_h.at[didx.at[j]],
                                 ddst.at[pl.ds(j * 128, 128)], sem)
                return c2
            lax.fori_loop(0, DPAD // 128, d_gat, 0)
            pltpu.make_async_copy(wd_h.at[pl.ds(0, DPAD)], ddst,
                                  sem).wait()
            pltpu.async_copy(
                ddst.at[pl.ds(0, DROWS)],
                outd_h.at[pl.ds((r0 + off) * NDF, DROWS)], sem).wait()
            return carry
        lax.fori_loop(0, NDCH, d_chunk, 0)

    return k(fsc_t, fdc_seg, ws_cols, wd_flat)


TB = 2


def _tc_assemble(stat_emb, stat_real, dyn_emb, dyn_real_nat):
    def body(se_ref, sr_ref, de_ref, dr_ref, o_ref):
        stat = jnp.concatenate([se_ref[...], sr_ref[...]], axis=-1)
        statb = jnp.broadcast_to(stat[None], (TB, B, CS + NRS))
        dr = jnp.transpose(dr_ref[...], (0, 2, 1))
        o_ref[...] = jnp.concatenate(
            [statb, de_ref[...], dr], axis=-1)

    return pl.pallas_call(
        body,
        compiler_params=pltpu.CompilerParams(
            vmem_limit_bytes=100 * 1024 * 1024),
        grid=(T // TB,),
        in_specs=[
            pl.BlockSpec((B, CS), lambda i: (0, 0)),
            pl.BlockSpec((B, NRS), lambda i: (0, 0)),
            pl.BlockSpec((TB, B, NDF * D), lambda i: (i, 0, 0)),
            pl.BlockSpec((TB, NRD, B), lambda i: (i, 0, 0)),
        ],
        out_specs=pl.BlockSpec((TB, B, COUT), lambda i: (i, 0, 0)),
        out_shape=jax.ShapeDtypeStruct((T, B, COUT), jnp.float32),
    )(stat_emb, stat_real, dyn_emb, dyn_real_nat)


def kernel(feat_static_cat, feat_static_real, feat_dynamic_cat,
           feat_dynamic_real, W_static, W_dynamic):
    ws_cols = jnp.transpose(W_static, (0, 2, 1)).reshape(NSF * D * V)
    wd_flat = W_dynamic.reshape(NDF * V, D)
    fsc_t = jnp.transpose(feat_static_cat.astype(jnp.int32),
                          (1, 0)).reshape(NSF * B)
    fdc_seg = jnp.transpose(feat_dynamic_cat.astype(jnp.int32),
                            (2, 1, 0)).reshape(NDF, T * B)
    out_stat, out_dyn = _sc_gather(fsc_t, fdc_seg, ws_cols, wd_flat)
    fdr_nat = jnp.transpose(feat_dynamic_real, (1, 2, 0))
    out_t = _tc_assemble(
        out_stat.reshape(B, CS),
        feat_static_real,
        out_dyn.reshape(T, B, NDF * D),
        fdr_nat,
    )
    return jnp.transpose(out_t, (1, 0, 2))
